# initial kernel scaffold (unmeasured)
import jax
import jax.numpy as jnp
from jax import lax
from jax.experimental import pallas as pl
from jax.experimental.pallas import tpu as pltpu

N_DEV = 4
SQ = 1024
SKV_SH = 1024
HQ = 8
DH = 128
D = HQ * DH
SCALE = 0.08838834764831843
NEG = jnp.float32(-1e9)


def _body(x_ref, wq_ref, k_ref, v_ref, wo_ref, out_ref,
          ctx_acc, ml_acc, ctx_comm, ml_comm,
          ctx_ssem, ctx_rsem, ml_ssem, ml_rsem):
    my_pos = lax.axis_index("i")
    left = lax.rem(my_pos + (N_DEV - 1), N_DEV)
    right = lax.rem(my_pos + 1, N_DEV)

    barrier_sem = pltpu.get_barrier_semaphore()
    for nbr in (left, right):
        pl.semaphore_signal(barrier_sem, inc=1, device_id=(nbr,),
                            device_id_type=pl.DeviceIdType.MESH)
    pl.semaphore_wait(barrier_sem, 2)

    q = jnp.dot(x_ref[...], wq_ref[...], preferred_element_type=jnp.float32)

    qi = lax.broadcasted_iota(jnp.int32, (SQ, SKV_SH), 0)
    kj = lax.broadcasted_iota(jnp.int32, (SQ, SKV_SH), 1)
    mask = ((qi // 64) % 4) == ((kj // 64) % 4)

    cs, ms, ls = [], [], []
    for h in range(HQ):
        qh = q[:, h * DH:(h + 1) * DH]
        kh = k_ref[:, h * DH:(h + 1) * DH]
        vh = v_ref[:, h * DH:(h + 1) * DH]
        s = lax.dot_general(qh, kh, (((1,), (1,)), ((), ())),
                            preferred_element_type=jnp.float32) * SCALE
        s = jnp.where(mask, s, NEG)
        m = jnp.max(s, axis=1, keepdims=True)
        w = jnp.exp(s - m)
        ls.append(jnp.sum(w, axis=1, keepdims=True))
        ms.append(m)
        cs.append(jnp.dot(w, vh, preferred_element_type=jnp.float32))
    ctx = jnp.concatenate(cs, axis=1)
    ml = jnp.concatenate(ms + ls, axis=1)
    ctx_acc[...] = ctx
    ml_acc[...] = ml
    ctx_comm[0] = ctx
    ml_comm[0] = ml

    for hop in range(N_DEV - 1):
        s_slot, r_slot = hop % 2, (hop + 1) % 2
        ctx_rdma = pltpu.make_async_remote_copy(
            src_ref=ctx_comm.at[s_slot], dst_ref=ctx_comm.at[r_slot],
            send_sem=ctx_ssem.at[hop], recv_sem=ctx_rsem.at[hop],
            device_id=(right,), device_id_type=pl.DeviceIdType.MESH)
        ml_rdma = pltpu.make_async_remote_copy(
            src_ref=ml_comm.at[s_slot], dst_ref=ml_comm.at[r_slot],
            send_sem=ml_ssem.at[hop], recv_sem=ml_rsem.at[hop],
            device_id=(right,), device_id_type=pl.DeviceIdType.MESH)
        ctx_rdma.start()
        ml_rdma.start()
        ctx_rdma.wait()
        ml_rdma.wait()

        ml_in = ml_comm[r_slot]
        m_in, l_in = ml_in[:, 0:HQ], ml_in[:, HQ:2 * HQ]
        ml_a = ml_acc[...]
        m_a, l_a = ml_a[:, 0:HQ], ml_a[:, HQ:2 * HQ]
        m_new = jnp.maximum(m_a, m_in)
        a = jnp.exp(m_a - m_new)
        b = jnp.exp(m_in - m_new)
        ml_acc[...] = jnp.concatenate([m_new, a * l_a + b * l_in], axis=1)
        ctx_in = ctx_comm[r_slot]
        for h in range(HQ):
            c0, c1 = h * DH, (h + 1) * DH
            ctx_acc[:, c0:c1] = (a[:, h:h + 1] * ctx_acc[:, c0:c1]
                                 + b[:, h:h + 1] * ctx_in[:, c0:c1])

    ml_f = ml_acc[...]
    inv_l = 1.0 / ml_f[:, HQ:2 * HQ]
    ctx_f = ctx_acc[...]
    normed = jnp.concatenate(
        [ctx_f[:, h * DH:(h + 1) * DH] * inv_l[:, h:h + 1]
         for h in range(HQ)], axis=1)
    out_ref[...] = jnp.dot(normed, wo_ref[...],
                           preferred_element_type=jnp.float32)


def kernel(x, Wq, K_ext, V_ext, Wo):
    x2 = x.reshape(SQ, D)
    k2 = K_ext.reshape(SKV_SH, D)
    v2 = V_ext.reshape(SKV_SH, D)

    out = pl.pallas_call(
        _body,
        out_shape=jax.ShapeDtypeStruct((SQ, D), jnp.float32),
        in_specs=[pl.BlockSpec(memory_space=pltpu.VMEM)] * 5,
        out_specs=pl.BlockSpec(memory_space=pltpu.VMEM),
        scratch_shapes=[
            pltpu.VMEM((SQ, D), jnp.float32),
            pltpu.VMEM((SQ, 2 * HQ), jnp.float32),
            pltpu.VMEM((2, SQ, D), jnp.float32),
            pltpu.VMEM((2, SQ, 2 * HQ), jnp.float32),
            pltpu.SemaphoreType.DMA((N_DEV - 1,)),
            pltpu.SemaphoreType.DMA((N_DEV - 1,)),
            pltpu.SemaphoreType.DMA((N_DEV - 1,)),
            pltpu.SemaphoreType.DMA((N_DEV - 1,)),
        ],
        compiler_params=pltpu.CompilerParams(collective_id=0),
    )(x2, Wq, k2, v2, Wo)
    return out.reshape(1, SQ, D)


# baseline (device time: 208317 ns/iter reference)
import jax
import jax.numpy as jnp
from jax import lax
from jax.experimental import pallas as pl
from jax.experimental.pallas import tpu as pltpu

N_DEV = 4
SQ = 1024
SKV_SH = 1024
HQ = 8
DH = 128
D = HQ * DH
SCALE = 0.08838834764831843
NEG = -1e9


def _body(x_ref, wq_ref, k_ref, v_ref, wo_ref, out_ref,
          ctx_acc, ml_acc, ctx_comm, ml_comm,
          ctx_ssem, ctx_rsem, ml_ssem, ml_rsem):
    my_pos = lax.axis_index("i")
    left = lax.rem(my_pos + (N_DEV - 1), N_DEV)
    right = lax.rem(my_pos + 1, N_DEV)

    barrier_sem = pltpu.get_barrier_semaphore()
    for nbr in (left, right):
        pl.semaphore_signal(barrier_sem, inc=1, device_id=(nbr,),
                            device_id_type=pl.DeviceIdType.MESH)
    pl.semaphore_wait(barrier_sem, 2)

    q = jnp.dot(x_ref[...], wq_ref[...], preferred_element_type=jnp.float32)

    qi = lax.broadcasted_iota(jnp.int32, (SQ, SKV_SH), 0)
    kj = lax.broadcasted_iota(jnp.int32, (SQ, SKV_SH), 1)
    mask = ((qi // 64) % 4) == ((kj // 64) % 4)

    cs, ms, ls = [], [], []
    for h in range(HQ):
        qh = q[:, h * DH:(h + 1) * DH]
        kh = k_ref[:, h * DH:(h + 1) * DH]
        vh = v_ref[:, h * DH:(h + 1) * DH]
        s = lax.dot_general(qh, kh, (((1,), (1,)), ((), ())),
                            preferred_element_type=jnp.float32) * SCALE
        s = jnp.where(mask, s, NEG)
        m = jnp.max(s, axis=1, keepdims=True)
        w = jnp.exp(s - m)
        ls.append(jnp.sum(w, axis=1, keepdims=True))
        ms.append(m)
        cs.append(jnp.dot(w, vh, preferred_element_type=jnp.float32))
    ctx = jnp.concatenate(cs, axis=1)
    ml = jnp.concatenate(ms + ls, axis=1)
    ctx_acc[...] = ctx
    ml_acc[...] = ml
    ctx_comm[0] = ctx
    ml_comm[0] = ml

    for hop in range(N_DEV - 1):
        s_slot, r_slot = hop % 2, (hop + 1) % 2
        ctx_rdma = pltpu.make_async_remote_copy(
            src_ref=ctx_comm.at[s_slot], dst_ref=ctx_comm.at[r_slot],
            send_sem=ctx_ssem.at[hop], recv_sem=ctx_rsem.at[hop],
            device_id=(right,), device_id_type=pl.DeviceIdType.MESH)
        ml_rdma = pltpu.make_async_remote_copy(
            src_ref=ml_comm.at[s_slot], dst_ref=ml_comm.at[r_slot],
            send_sem=ml_ssem.at[hop], recv_sem=ml_rsem.at[hop],
            device_id=(right,), device_id_type=pl.DeviceIdType.MESH)
        ctx_rdma.start()
        ml_rdma.start()
        ctx_rdma.wait()
        ml_rdma.wait()

        ml_in = ml_comm[r_slot]
        m_in, l_in = ml_in[:, 0:HQ], ml_in[:, HQ:2 * HQ]
        ml_a = ml_acc[...]
        m_a, l_a = ml_a[:, 0:HQ], ml_a[:, HQ:2 * HQ]
        m_new = jnp.maximum(m_a, m_in)
        a = jnp.exp(m_a - m_new)
        b = jnp.exp(m_in - m_new)
        ml_acc[...] = jnp.concatenate([m_new, a * l_a + b * l_in], axis=1)
        ctx_in = ctx_comm[r_slot]
        for h in range(HQ):
            c0, c1 = h * DH, (h + 1) * DH
            ctx_acc[:, c0:c1] = (a[:, h:h + 1] * ctx_acc[:, c0:c1]
                                 + b[:, h:h + 1] * ctx_in[:, c0:c1])

    ml_f = ml_acc[...]
    inv_l = 1.0 / ml_f[:, HQ:2 * HQ]
    ctx_f = ctx_acc[...]
    normed = jnp.concatenate(
        [ctx_f[:, h * DH:(h + 1) * DH] * inv_l[:, h:h + 1]
         for h in range(HQ)], axis=1)
    out_ref[...] = jnp.dot(normed, wo_ref[...],
                           preferred_element_type=jnp.float32)


def kernel(x, Wq, K_ext, V_ext, Wo):
    x2 = x.reshape(SQ, D)
    k2 = K_ext.reshape(SKV_SH, D)
    v2 = V_ext.reshape(SKV_SH, D)

    out = pl.pallas_call(
        _body,
        out_shape=jax.ShapeDtypeStruct((SQ, D), jnp.float32),
        in_specs=[pl.BlockSpec(memory_space=pltpu.VMEM)] * 5,
        out_specs=pl.BlockSpec(memory_space=pltpu.VMEM),
        scratch_shapes=[
            pltpu.VMEM((SQ, D), jnp.float32),
            pltpu.VMEM((SQ, 2 * HQ), jnp.float32),
            pltpu.VMEM((2, SQ, D), jnp.float32),
            pltpu.VMEM((2, SQ, 2 * HQ), jnp.float32),
            pltpu.SemaphoreType.DMA((N_DEV - 1,)),
            pltpu.SemaphoreType.DMA((N_DEV - 1,)),
            pltpu.SemaphoreType.DMA((N_DEV - 1,)),
            pltpu.SemaphoreType.DMA((N_DEV - 1,)),
        ],
        compiler_params=pltpu.CompilerParams(
            collective_id=0, vmem_limit_bytes=100 * 1024 * 1024),
    )(x2, Wq, k2, v2, Wo)
    return out.reshape(1, SQ, D)


# device time: 94534 ns/iter; 2.2036x vs baseline; 2.2036x over previous
import jax
import jax.numpy as jnp
from jax import lax
from jax.experimental import pallas as pl
from jax.experimental.pallas import tpu as pltpu

N_DEV = 4
SQ = 1024
SKV_SH = 1024
HQ = 8
DH = 128
D = HQ * DH
CHUNK = SQ // N_DEV
SCALE = 0.08838834764831843
NEG = -1e9


def _body(x_ref, wq_ref, k_ref, v_ref, wo_ref, out_ref,
          ctx_acc, ml_acc, rs_ctx, rs_ml,
          rs_ctx_ssem, rs_ctx_rsem, rs_ml_ssem, rs_ml_rsem,
          ag_ssem, ag_rsem):
    my_pos = lax.axis_index("i")

    barrier_sem = pltpu.get_barrier_semaphore()
    for dlt in (1, 2, 3):
        nbr = lax.rem(my_pos + dlt, N_DEV)
        pl.semaphore_signal(barrier_sem, inc=1, device_id=(nbr,),
                            device_id_type=pl.DeviceIdType.MESH)
    pl.semaphore_wait(barrier_sem, N_DEV - 1)

    q = jnp.dot(x_ref[...], wq_ref[...], preferred_element_type=jnp.float32)

    qi = lax.broadcasted_iota(jnp.int32, (SQ, SKV_SH), 0)
    kj = lax.broadcasted_iota(jnp.int32, (SQ, SKV_SH), 1)
    mask = ((qi // 64) % 4) == ((kj // 64) % 4)

    cs, ms, ls = [], [], []
    for h in range(HQ):
        qh = q[:, h * DH:(h + 1) * DH]
        kh = k_ref[:, h * DH:(h + 1) * DH]
        vh = v_ref[:, h * DH:(h + 1) * DH]
        s = lax.dot_general(qh, kh, (((1,), (1,)), ((), ())),
                            preferred_element_type=jnp.float32) * SCALE
        s = jnp.where(mask, s, NEG)
        m = jnp.max(s, axis=1, keepdims=True)
        w = jnp.exp(s - m)
        ls.append(jnp.sum(w, axis=1, keepdims=True))
        ms.append(m)
        cs.append(jnp.dot(w, vh, preferred_element_type=jnp.float32))
    ctx_acc[...] = jnp.concatenate(cs, axis=1)
    ml_acc[...] = jnp.concatenate(ms + ls, axis=1)

    rs_sends = []
    for dlt in (1, 2, 3):
        tgt = lax.rem(my_pos + dlt, N_DEV)
        t_off = tgt * CHUNK
        k_slot = dlt - 1
        ctx_rdma = pltpu.make_async_remote_copy(
            src_ref=ctx_acc.at[pl.ds(t_off, CHUNK)],
            dst_ref=rs_ctx.at[k_slot],
            send_sem=rs_ctx_ssem.at[k_slot], recv_sem=rs_ctx_rsem.at[k_slot],
            device_id=(tgt,), device_id_type=pl.DeviceIdType.MESH)
        ml_rdma = pltpu.make_async_remote_copy(
            src_ref=ml_acc.at[pl.ds(t_off, CHUNK)],
            dst_ref=rs_ml.at[k_slot],
            send_sem=rs_ml_ssem.at[k_slot], recv_sem=rs_ml_rsem.at[k_slot],
            device_id=(tgt,), device_id_type=pl.DeviceIdType.MESH)
        ctx_rdma.start()
        ml_rdma.start()
        rs_sends.append((ctx_rdma, ml_rdma))

    for ctx_rdma, ml_rdma in rs_sends:
        ctx_rdma.wait_recv()
        ml_rdma.wait_recv()

    off = my_pos * CHUNK
    ml_own = ml_acc[pl.ds(off, CHUNK), :]
    m_parts = [ml_own[:, 0:HQ]] + [rs_ml[k][:, 0:HQ] for k in range(3)]
    l_parts = [ml_own[:, HQ:2 * HQ]] + [rs_ml[k][:, HQ:2 * HQ]
                                        for k in range(3)]
    c_parts = [ctx_acc[pl.ds(off, CHUNK), :]] + [rs_ctx[k] for k in range(3)]

    m_tot = jnp.maximum(jnp.maximum(m_parts[0], m_parts[1]),
                        jnp.maximum(m_parts[2], m_parts[3]))
    scales = [jnp.exp(mp - m_tot) for mp in m_parts]
    l_tot = sum(s * lp for s, lp in zip(scales, l_parts))
    inv_l = 1.0 / l_tot
    cols = []
    for h in range(HQ):
        c0, c1 = h * DH, (h + 1) * DH
        acc = scales[0][:, h:h + 1] * c_parts[0][:, c0:c1]
        for k in range(1, 4):
            acc = acc + scales[k][:, h:h + 1] * c_parts[k][:, c0:c1]
        cols.append(acc * inv_l[:, h:h + 1])
    ctx_chunk = jnp.concatenate(cols, axis=1)

    out_ref[pl.ds(off, CHUNK), :] = jnp.dot(
        ctx_chunk, wo_ref[...], preferred_element_type=jnp.float32)

    ag_sends = []
    for dlt in (1, 2, 3):
        tgt = lax.rem(my_pos + dlt, N_DEV)
        k_slot = dlt - 1
        ag_rdma = pltpu.make_async_remote_copy(
            src_ref=out_ref.at[pl.ds(off, CHUNK)],
            dst_ref=out_ref.at[pl.ds(off, CHUNK)],
            send_sem=ag_ssem.at[k_slot], recv_sem=ag_rsem.at[k_slot],
            device_id=(tgt,), device_id_type=pl.DeviceIdType.MESH)
        ag_rdma.start()
        ag_sends.append(ag_rdma)

    for ctx_rdma, ml_rdma in rs_sends:
        ctx_rdma.wait_send()
        ml_rdma.wait_send()
    for ag_rdma in ag_sends:
        ag_rdma.wait_recv()
    for ag_rdma in ag_sends:
        ag_rdma.wait_send()


def kernel(x, Wq, K_ext, V_ext, Wo):
    x2 = x.reshape(SQ, D)
    k2 = K_ext.reshape(SKV_SH, D)
    v2 = V_ext.reshape(SKV_SH, D)

    out = pl.pallas_call(
        _body,
        out_shape=jax.ShapeDtypeStruct((SQ, D), jnp.float32),
        in_specs=[pl.BlockSpec(memory_space=pltpu.VMEM)] * 5,
        out_specs=pl.BlockSpec(memory_space=pltpu.VMEM),
        scratch_shapes=[
            pltpu.VMEM((SQ, D), jnp.float32),
            pltpu.VMEM((SQ, 2 * HQ), jnp.float32),
            pltpu.VMEM((3, CHUNK, D), jnp.float32),
            pltpu.VMEM((3, CHUNK, 2 * HQ), jnp.float32),
            pltpu.SemaphoreType.DMA((3,)),
            pltpu.SemaphoreType.DMA((3,)),
            pltpu.SemaphoreType.DMA((3,)),
            pltpu.SemaphoreType.DMA((3,)),
            pltpu.SemaphoreType.DMA((3,)),
            pltpu.SemaphoreType.DMA((3,)),
        ],
        compiler_params=pltpu.CompilerParams(
            collective_id=0, vmem_limit_bytes=100 * 1024 * 1024),
    )(x2, Wq, k2, v2, Wo)
    return out.reshape(1, SQ, D)


# device time: 63324 ns/iter; 3.2897x vs baseline; 1.4929x over previous
import jax
import jax.numpy as jnp
from jax import lax
from jax.experimental import pallas as pl
from jax.experimental.pallas import tpu as pltpu

N_DEV = 4
SQ = 1024
SKV_SH = 1024
HQ = 8
DH = 128
D = HQ * DH
CHUNK = SQ // N_DEV
SCALE = 0.08838834764831843
NEG = -1e9


def _body(x_ref, wq_ref, k_ref, v_ref, wo_ref, out_ref,
          ctx_acc, ml_acc, rs_ctx, rs_ml, ag_buf,
          rs_ctx_ssem, rs_ctx_rsem, rs_ml_ssem, rs_ml_rsem,
          ag_ssem, ag_rsem):
    my_pos = lax.axis_index("i")

    barrier_sem = pltpu.get_barrier_semaphore()
    for dlt in (1, 2, 3):
        nbr = lax.rem(my_pos + dlt, N_DEV)
        pl.semaphore_signal(barrier_sem, inc=1, device_id=(nbr,),
                            device_id_type=pl.DeviceIdType.MESH)

    qi = lax.broadcasted_iota(jnp.int32, (CHUNK, SKV_SH), 0)
    kj = lax.broadcasted_iota(jnp.int32, (CHUNK, SKV_SH), 1)
    mask = ((qi // 64) % 4) == ((kj // 64) % 4)

    barrier_waited = False
    rs_sends = []
    for dlt in (1, 2, 3, 0):
        tgt = lax.rem(my_pos + dlt, N_DEV)
        t_off = tgt * CHUNK
        qc = jnp.dot(x_ref[pl.ds(t_off, CHUNK), :], wq_ref[...],
                     preferred_element_type=jnp.float32)
        cs, ms, ls = [], [], []
        for h in range(HQ):
            qh = qc[:, h * DH:(h + 1) * DH]
            kh = k_ref[:, h * DH:(h + 1) * DH]
            vh = v_ref[:, h * DH:(h + 1) * DH]
            s = lax.dot_general(qh, kh, (((1,), (1,)), ((), ())),
                                preferred_element_type=jnp.float32) * SCALE
            s = jnp.where(mask, s, NEG)
            m = jnp.max(s, axis=1, keepdims=True)
            w = jnp.exp(s - m)
            ls.append(jnp.sum(w, axis=1, keepdims=True))
            ms.append(m)
            cs.append(jnp.dot(w, vh, preferred_element_type=jnp.float32))
        ctx_acc[pl.ds(t_off, CHUNK), :] = jnp.concatenate(
            cs, axis=1).astype(jnp.bfloat16)
        ml_acc[pl.ds(t_off, CHUNK), :] = jnp.concatenate(ms + ls, axis=1)

        if dlt != 0:
            if not barrier_waited:
                pl.semaphore_wait(barrier_sem, N_DEV - 1)
                barrier_waited = True
            k_slot = dlt - 1
            ctx_rdma = pltpu.make_async_remote_copy(
                src_ref=ctx_acc.at[pl.ds(t_off, CHUNK)],
                dst_ref=rs_ctx.at[k_slot],
                send_sem=rs_ctx_ssem.at[k_slot],
                recv_sem=rs_ctx_rsem.at[k_slot],
                device_id=(tgt,), device_id_type=pl.DeviceIdType.MESH)
            ml_rdma = pltpu.make_async_remote_copy(
                src_ref=ml_acc.at[pl.ds(t_off, CHUNK)],
                dst_ref=rs_ml.at[k_slot],
                send_sem=rs_ml_ssem.at[k_slot],
                recv_sem=rs_ml_rsem.at[k_slot],
                device_id=(tgt,), device_id_type=pl.DeviceIdType.MESH)
            ctx_rdma.start()
            ml_rdma.start()
            rs_sends.append((ctx_rdma, ml_rdma))

    for ctx_rdma, ml_rdma in rs_sends:
        ctx_rdma.wait_recv()
        ml_rdma.wait_recv()

    off = my_pos * CHUNK
    ml_own = ml_acc[pl.ds(off, CHUNK), :]
    m_parts = [ml_own[:, 0:HQ]] + [rs_ml[k][:, 0:HQ] for k in range(3)]
    l_parts = [ml_own[:, HQ:2 * HQ]] + [rs_ml[k][:, HQ:2 * HQ]
                                        for k in range(3)]
    c_parts = [ctx_acc[pl.ds(off, CHUNK), :].astype(jnp.float32)] + [
        rs_ctx[k].astype(jnp.float32) for k in range(3)]

    m_tot = jnp.maximum(jnp.maximum(m_parts[0], m_parts[1]),
                        jnp.maximum(m_parts[2], m_parts[3]))
    scales = [jnp.exp(mp - m_tot) for mp in m_parts]
    l_tot = sum(s * lp for s, lp in zip(scales, l_parts))
    inv_l = 1.0 / l_tot
    cols = []
    for h in range(HQ):
        c0, c1 = h * DH, (h + 1) * DH
        acc = scales[0][:, h:h + 1] * c_parts[0][:, c0:c1]
        for k in range(1, 4):
            acc = acc + scales[k][:, h:h + 1] * c_parts[k][:, c0:c1]
        cols.append(acc * inv_l[:, h:h + 1])
    ctx_chunk = jnp.concatenate(cols, axis=1)

    out_chunk = jnp.dot(ctx_chunk, wo_ref[...],
                        preferred_element_type=jnp.float32)
    ag_buf[pl.ds(off, CHUNK), :] = out_chunk.astype(jnp.bfloat16)

    ag_sends = []
    for dlt in (1, 2, 3):
        tgt = lax.rem(my_pos + dlt, N_DEV)
        k_slot = dlt - 1
        ag_rdma = pltpu.make_async_remote_copy(
            src_ref=ag_buf.at[pl.ds(off, CHUNK)],
            dst_ref=ag_buf.at[pl.ds(off, CHUNK)],
            send_sem=ag_ssem.at[k_slot], recv_sem=ag_rsem.at[k_slot],
            device_id=(tgt,), device_id_type=pl.DeviceIdType.MESH)
        ag_rdma.start()
        ag_sends.append(ag_rdma)

    for ctx_rdma, ml_rdma in rs_sends:
        ctx_rdma.wait_send()
        ml_rdma.wait_send()
    for ag_rdma in ag_sends:
        ag_rdma.wait_recv()
    for ag_rdma in ag_sends:
        ag_rdma.wait_send()

    out_ref[...] = ag_buf[...].astype(jnp.float32)


def kernel(x, Wq, K_ext, V_ext, Wo):
    x2 = x.reshape(SQ, D)
    k2 = K_ext.reshape(SKV_SH, D)
    v2 = V_ext.reshape(SKV_SH, D)

    out = pl.pallas_call(
        _body,
        out_shape=jax.ShapeDtypeStruct((SQ, D), jnp.float32),
        in_specs=[pl.BlockSpec(memory_space=pltpu.VMEM)] * 5,
        out_specs=pl.BlockSpec(memory_space=pltpu.VMEM),
        scratch_shapes=[
            pltpu.VMEM((SQ, D), jnp.bfloat16),
            pltpu.VMEM((SQ, 2 * HQ), jnp.float32),
            pltpu.VMEM((3, CHUNK, D), jnp.bfloat16),
            pltpu.VMEM((3, CHUNK, 2 * HQ), jnp.float32),
            pltpu.VMEM((SQ, D), jnp.bfloat16),
            pltpu.SemaphoreType.DMA((3,)),
            pltpu.SemaphoreType.DMA((3,)),
            pltpu.SemaphoreType.DMA((3,)),
            pltpu.SemaphoreType.DMA((3,)),
            pltpu.SemaphoreType.DMA((3,)),
            pltpu.SemaphoreType.DMA((3,)),
        ],
        compiler_params=pltpu.CompilerParams(
            collective_id=0, vmem_limit_bytes=100 * 1024 * 1024),
    )(x2, Wq, k2, v2, Wo)
    return out.reshape(1, SQ, D)


# device time: 58959 ns/iter; 3.5333x vs baseline; 1.0740x over previous
import jax
import jax.numpy as jnp
from jax import lax
from jax.experimental import pallas as pl
from jax.experimental.pallas import tpu as pltpu

N_DEV = 4
SQ = 1024
SKV_SH = 1024
HQ = 8
DH = 128
D = HQ * DH
CHUNK = SQ // N_DEV
BLK = 64
SCALE = 0.08838834764831843


def _gather_group(ref, g):
    return jnp.concatenate(
        [ref[pl.ds(BLK * g + CHUNK * t, BLK), :] for t in range(4)], axis=0)


def _body(x_ref, wq_ref, k_ref, v_ref, wo_ref, out_ref,
          ctx_acc, ml_acc, rs_ctx, rs_ml, ag_buf,
          rs_ctx_ssem, rs_ctx_rsem, rs_ml_ssem, rs_ml_rsem,
          ag_ssem, ag_rsem):
    my_pos = lax.axis_index("i")

    barrier_sem = pltpu.get_barrier_semaphore()
    for dlt in (1, 2, 3):
        nbr = lax.rem(my_pos + dlt, N_DEV)
        pl.semaphore_signal(barrier_sem, inc=1, device_id=(nbr,),
                            device_id_type=pl.DeviceIdType.MESH)

    wq_bf = wq_ref[...].astype(jnp.bfloat16)

    barrier_waited = False
    rs_sends = []
    for dlt in (1, 2, 3, 0):
        tgt = lax.rem(my_pos + dlt, N_DEV)
        t_off = tgt * CHUNK
        xg = _gather_group(x_ref, tgt).astype(jnp.bfloat16)
        kg = _gather_group(k_ref, tgt).astype(jnp.bfloat16)
        vg = _gather_group(v_ref, tgt).astype(jnp.bfloat16)
        qg = jnp.dot(xg, wq_bf, preferred_element_type=jnp.float32)
        qg = qg.astype(jnp.bfloat16)

        cs, ms, ls = [], [], []
        for h in range(HQ):
            c0, c1 = h * DH, (h + 1) * DH
            s = lax.dot_general(qg[:, c0:c1], kg[:, c0:c1],
                                (((1,), (1,)), ((), ())),
                                preferred_element_type=jnp.float32) * SCALE
            m = jnp.max(s, axis=1, keepdims=True)
            w = jnp.exp(s - m)
            ls.append(jnp.sum(w, axis=1, keepdims=True))
            ms.append(m)
            cs.append(jnp.dot(w.astype(jnp.bfloat16), vg[:, c0:c1],
                              preferred_element_type=jnp.float32))
        ctx_acc[pl.ds(t_off, CHUNK), :] = jnp.concatenate(
            cs, axis=1).astype(jnp.bfloat16)
        ml_acc[pl.ds(t_off, CHUNK), :] = jnp.concatenate(ms + ls, axis=1)

        if dlt != 0:
            if not barrier_waited:
                pl.semaphore_wait(barrier_sem, N_DEV - 1)
                barrier_waited = True
            k_slot = dlt - 1
            ctx_rdma = pltpu.make_async_remote_copy(
                src_ref=ctx_acc.at[pl.ds(t_off, CHUNK)],
                dst_ref=rs_ctx.at[k_slot],
                send_sem=rs_ctx_ssem.at[k_slot],
                recv_sem=rs_ctx_rsem.at[k_slot],
                device_id=(tgt,), device_id_type=pl.DeviceIdType.MESH)
            ml_rdma = pltpu.make_async_remote_copy(
                src_ref=ml_acc.at[pl.ds(t_off, CHUNK)],
                dst_ref=rs_ml.at[k_slot],
                send_sem=rs_ml_ssem.at[k_slot],
                recv_sem=rs_ml_rsem.at[k_slot],
                device_id=(tgt,), device_id_type=pl.DeviceIdType.MESH)
            ctx_rdma.start()
            ml_rdma.start()
            rs_sends.append((ctx_rdma, ml_rdma))

    for ctx_rdma, ml_rdma in rs_sends:
        ctx_rdma.wait_recv()
        ml_rdma.wait_recv()

    off = my_pos * CHUNK
    ml_own = ml_acc[pl.ds(off, CHUNK), :]
    m_parts = [ml_own[:, 0:HQ]] + [rs_ml[k][:, 0:HQ] for k in range(3)]
    l_parts = [ml_own[:, HQ:2 * HQ]] + [rs_ml[k][:, HQ:2 * HQ]
                                        for k in range(3)]
    c_parts = [ctx_acc[pl.ds(off, CHUNK), :].astype(jnp.float32)] + [
        rs_ctx[k].astype(jnp.float32) for k in range(3)]

    m_tot = jnp.maximum(jnp.maximum(m_parts[0], m_parts[1]),
                        jnp.maximum(m_parts[2], m_parts[3]))
    scales = [jnp.exp(mp - m_tot) for mp in m_parts]
    l_tot = sum(s * lp for s, lp in zip(scales, l_parts))
    inv_l = 1.0 / l_tot
    cols = []
    for h in range(HQ):
        c0, c1 = h * DH, (h + 1) * DH
        acc = scales[0][:, h:h + 1] * c_parts[0][:, c0:c1]
        for k in range(1, 4):
            acc = acc + scales[k][:, h:h + 1] * c_parts[k][:, c0:c1]
        cols.append(acc * inv_l[:, h:h + 1])
    ctx_chunk = jnp.concatenate(cols, axis=1)

    out_chunk = jnp.dot(ctx_chunk.astype(jnp.bfloat16),
                        wo_ref[...].astype(jnp.bfloat16),
                        preferred_element_type=jnp.float32)
    ag_buf[pl.ds(off, CHUNK), :] = out_chunk.astype(jnp.bfloat16)

    ag_sends = []
    for dlt in (1, 2, 3):
        tgt = lax.rem(my_pos + dlt, N_DEV)
        k_slot = dlt - 1
        ag_rdma = pltpu.make_async_remote_copy(
            src_ref=ag_buf.at[pl.ds(off, CHUNK)],
            dst_ref=ag_buf.at[pl.ds(off, CHUNK)],
            send_sem=ag_ssem.at[k_slot], recv_sem=ag_rsem.at[k_slot],
            device_id=(tgt,), device_id_type=pl.DeviceIdType.MESH)
        ag_rdma.start()
        ag_sends.append(ag_rdma)

    for ctx_rdma, ml_rdma in rs_sends:
        ctx_rdma.wait_send()
        ml_rdma.wait_send()
    for ag_rdma in ag_sends:
        ag_rdma.wait_recv()
    for ag_rdma in ag_sends:
        ag_rdma.wait_send()

    for g in range(N_DEV):
        for t in range(4):
            out_ref[BLK * g + CHUNK * t:BLK * g + CHUNK * t + BLK, :] = (
                ag_buf[CHUNK * g + BLK * t:CHUNK * g + BLK * t + BLK, :]
                .astype(jnp.float32))


def kernel(x, Wq, K_ext, V_ext, Wo):
    x2 = x.reshape(SQ, D)
    k2 = K_ext.reshape(SKV_SH, D)
    v2 = V_ext.reshape(SKV_SH, D)

    out = pl.pallas_call(
        _body,
        out_shape=jax.ShapeDtypeStruct((SQ, D), jnp.float32),
        in_specs=[pl.BlockSpec(memory_space=pltpu.VMEM)] * 5,
        out_specs=pl.BlockSpec(memory_space=pltpu.VMEM),
        scratch_shapes=[
            pltpu.VMEM((SQ, D), jnp.bfloat16),
            pltpu.VMEM((SQ, 2 * HQ), jnp.float32),
            pltpu.VMEM((3, CHUNK, D), jnp.bfloat16),
            pltpu.VMEM((3, CHUNK, 2 * HQ), jnp.float32),
            pltpu.VMEM((SQ, D), jnp.bfloat16),
            pltpu.SemaphoreType.DMA((3,)),
            pltpu.SemaphoreType.DMA((3,)),
            pltpu.SemaphoreType.DMA((3,)),
            pltpu.SemaphoreType.DMA((3,)),
            pltpu.SemaphoreType.DMA((3,)),
            pltpu.SemaphoreType.DMA((3,)),
        ],
        compiler_params=pltpu.CompilerParams(
            collective_id=0, vmem_limit_bytes=100 * 1024 * 1024),
    )(x2, Wq, k2, v2, Wo)
    return out.reshape(1, SQ, D)
